# baseline (speedup 1.0000x reference)
import functools

import jax
import jax.numpy as jnp
from jax import lax
from jax.experimental import pallas as pl
from jax.experimental.pallas import tpu as pltpu
from jax.experimental.pallas import tpu_sc as plsc

N = 10000
D = 128
E = 320000
NC = 2
NS = 16
NW = NC * NS
CH = 128
G = 16
K2 = 5
K = K2 * G
EPW = K * CH
E_PAD = NW * EPW
NROW = 10240
ZR = NROW // NS


def _sc_agg_body(x_hbm, src_hbm, dst_hbm, out_hbm, sslab, dslab, bufs,
                 agg_sh, isem, gsems, ssems):
    c = lax.axis_index("c")
    s = lax.axis_index("s")
    wid = s * NC + c

    z16 = jnp.zeros((16,), jnp.float32)

    @pl.loop(0, CH)
    def _zero_row(i):
        for cc in range(D // 16):
            bufs[0][i, pl.ds(cc * 16, 16)] = z16

    zbase = s * ZR
    for t in range(ZR // CH):
        pltpu.async_copy(bufs[0], agg_sh.at[pl.ds(zbase + t * CH, CH)],
                         gsems[1])

    row0 = wid * (K2 * G)
    pltpu.async_copy(src_hbm.at[pl.ds(row0, G)], sslab.at[0], isem)
    pltpu.async_copy(dst_hbm.at[pl.ds(row0, G)], dslab.at[0], isem)
    pltpu.async_copy(src_hbm.at[pl.ds(row0 + G, G)], sslab.at[1], isem)
    pltpu.async_copy(dst_hbm.at[pl.ds(row0 + G, G)], dslab.at[1], isem)
    pltpu.make_async_copy(src_hbm.at[pl.ds(row0, G)], sslab.at[0], isem).wait()
    pltpu.make_async_copy(dst_hbm.at[pl.ds(row0, G)], dslab.at[0], isem).wait()

    for t in range(ZR // CH):
        pltpu.make_async_copy(bufs[0], agg_sh.at[pl.ds(zbase + t * CH, CH)],
                              gsems[1]).wait()
    pltpu.async_copy(x_hbm.at[sslab.at[0, 0]], bufs[0], gsems[0])
    plsc.subcore_barrier()

    def slot(g, p, q, u, first_group, last_group):
        b = u % 2
        pltpu.make_async_copy(x_hbm.at[sslab.at[p, u]], bufs[b],
                              gsems[b]).wait()
        pltpu.async_copy(bufs[b], agg_sh.at[dslab.at[p, u]], ssems[b],
                         add=True)
        if not (first_group and u == 0):
            prev = dslab.at[q, G - 1] if u == 0 else dslab.at[p, u - 1]
            pltpu.make_async_copy(bufs[1 - b], agg_sh.at[prev],
                                  ssems[1 - b]).wait()
        if u == 0 and not first_group and not last_group:
            pltpu.async_copy(src_hbm.at[pl.ds(row0 + (g + 1) * G, G)],
                             sslab.at[q], isem)
            pltpu.async_copy(dst_hbm.at[pl.ds(row0 + (g + 1) * G, G)],
                             dslab.at[q], isem)
        if not (last_group and u == G - 1):
            if u == G - 1:
                pltpu.make_async_copy(src_hbm.at[pl.ds(row0 + (g + 1) * G, G)],
                                      sslab.at[q], isem).wait()
                pltpu.make_async_copy(dst_hbm.at[pl.ds(row0 + (g + 1) * G, G)],
                                      dslab.at[q], isem).wait()
                nxt = sslab.at[q, 0]
            else:
                nxt = sslab.at[p, u + 1]
            pltpu.async_copy(x_hbm.at[nxt], bufs[1 - b], gsems[1 - b])

    def group(g, p, q, first_group=False, last_group=False):
        for u in range(G):
            slot(g, p, q, u, first_group, last_group)

    group(0, 0, 1, first_group=True)

    @pl.loop(1, K2 - 1)
    def _grp(g):
        p = g & 1
        group(g, p, 1 - p)

    group(K2 - 1, (K2 - 1) & 1, 1 - ((K2 - 1) & 1), last_group=True)
    pltpu.make_async_copy(bufs[(G - 1) % 2],
                          agg_sh.at[dslab.at[(K2 - 1) & 1, G - 1]],
                          ssems[(G - 1) % 2]).wait()

    plsc.subcore_barrier()

    pltpu.async_copy(agg_sh.at[pl.ds(zbase, ZR)],
                     out_hbm.at[c, pl.ds(zbase, ZR)], ssems[0])
    pltpu.make_async_copy(agg_sh.at[pl.ds(zbase, ZR)],
                          out_hbm.at[c, pl.ds(zbase, ZR)], ssems[0]).wait()


@functools.cache
def _make_sc_agg():
    return pl.kernel(
        _sc_agg_body,
        out_type=jax.ShapeDtypeStruct((NC, NROW, D), jnp.float32),
        mesh=plsc.VectorSubcoreMesh(core_axis_name="c", subcore_axis_name="s",
                                    num_cores=NC, num_subcores=NS),
        scratch_types=[
            pltpu.VMEM((2, G, CH), jnp.int32),
            pltpu.VMEM((2, G, CH), jnp.int32),
            [pltpu.VMEM((CH, D), jnp.float32) for _ in range(2)],
            pltpu.VMEM_SHARED((NROW, D), jnp.float32),
            pltpu.SemaphoreType.DMA,
            [pltpu.SemaphoreType.DMA for _ in range(2)],
            [pltpu.SemaphoreType.DMA for _ in range(2)],
        ],
    )


def _sc_agg(x, src_w, dst_w):
    return _make_sc_agg()(x, src_w, dst_w)


_PBR = 320


def _prep_body(ei_ref, src_ref, dst_ref):
    i = pl.program_id(0)
    eloc = (jax.lax.broadcasted_iota(jnp.int32, (_PBR, CH), 0) * CH
            + jax.lax.broadcasted_iota(jnp.int32, (_PBR, CH), 1))
    e = i * (_PBR * CH) + eloc
    real = e < E
    ei = ei_ref[...]
    src_ref[...] = jnp.where(real, ei[0].reshape(_PBR, CH), e % N)
    dst_ref[...] = jnp.where(real, ei[1].reshape(_PBR, CH),
                             N + e % (NROW - N))


def _prep_edges(edge_index):
    nchunk = E_PAD // CH
    return pl.pallas_call(
        _prep_body,
        grid=(nchunk // _PBR,),
        in_specs=[pl.BlockSpec((2, _PBR * CH), lambda i: (0, i))],
        out_specs=[pl.BlockSpec((_PBR, CH), lambda i: (i, 0)),
                   pl.BlockSpec((_PBR, CH), lambda i: (i, 0))],
        out_shape=[jax.ShapeDtypeStruct((nchunk, CH), jnp.int32),
                   jax.ShapeDtypeStruct((nchunk, CH), jnp.int32)],
    )(edge_index)


def _mlp_hidden_body(x_ref, p_ref, wa_ref, ba_ref, wb_ref, bb_ref, o_ref):
    p = p_ref[...]
    h = x_ref[...] + p[0] + p[1]
    t = jnp.maximum(
        jnp.dot(h, wa_ref[...], preferred_element_type=jnp.float32)
        + ba_ref[...], 0.0)
    o_ref[...] = jnp.maximum(
        jnp.dot(t, wb_ref[...], preferred_element_type=jnp.float32)
        + bb_ref[...], 0.0)


def _mlp_final_body(x_ref, p_ref, wa_ref, ba_ref, wb_ref, bb_ref,
                    wfc_ref, bfc_ref, o_ref):
    p = p_ref[...]
    h = x_ref[...] + p[0] + p[1]
    t = jnp.maximum(
        jnp.dot(h, wa_ref[...], preferred_element_type=jnp.float32)
        + ba_ref[...], 0.0)
    t = jnp.maximum(
        jnp.dot(t, wb_ref[...], preferred_element_type=jnp.float32)
        + bb_ref[...], 0.0)
    o_ref[...] = jax.nn.sigmoid(
        jnp.dot(t, wfc_ref[...], preferred_element_type=jnp.float32)
        + bfc_ref[...])


_RB = 2000


def _w_spec(d0, d1):
    return pl.BlockSpec((d0, d1), lambda i: (0, 0))


def _mlp_hidden(x, p, wa, ba, wb, bb):
    return pl.pallas_call(
        _mlp_hidden_body,
        grid=(N // _RB,),
        in_specs=[
            pl.BlockSpec((_RB, D), lambda i: (i, 0)),
            pl.BlockSpec((NC, _RB, D), lambda i: (0, i, 0)),
            _w_spec(D, D), _w_spec(1, D), _w_spec(D, D), _w_spec(1, D),
        ],
        out_specs=pl.BlockSpec((_RB, D), lambda i: (i, 0)),
        out_shape=jax.ShapeDtypeStruct((N, D), jnp.float32),
    )(x, p, wa, ba, wb, bb)


def _mlp_final(x, p, wa, ba, wb, bb, wfc, bfc):
    dout = wfc.shape[1]
    return pl.pallas_call(
        _mlp_final_body,
        grid=(N // _RB,),
        in_specs=[
            pl.BlockSpec((_RB, D), lambda i: (i, 0)),
            pl.BlockSpec((NC, _RB, D), lambda i: (0, i, 0)),
            _w_spec(D, D), _w_spec(1, D), _w_spec(D, D), _w_spec(1, D),
            _w_spec(D, dout), _w_spec(1, dout),
        ],
        out_specs=pl.BlockSpec((_RB, dout), lambda i: (i, 0)),
        out_shape=jax.ShapeDtypeStruct((N, dout), jnp.float32),
    )(x, p, wa, ba, wb, bb, wfc, bfc)


def kernel(x, edge_index, W1a, b1a, W1b, b1b, W2a, b2a, W2b, b2b, Wfc, bfc):
    src_w, dst_w = _prep_edges(edge_index)

    b1a2, b1b2 = b1a.reshape(1, D), b1b.reshape(1, D)
    b2a2, b2b2 = b2a.reshape(1, D), b2b.reshape(1, D)
    bfc2 = bfc.reshape(1, -1)

    p1 = _sc_agg(x, src_w, dst_w)
    h1 = _mlp_hidden(x, p1, W1a, b1a2, W1b, b1b2)
    p2 = _sc_agg(h1, src_w, dst_w)
    return _mlp_final(h1, p2, W2a, b2a2, W2b, b2b2, Wfc, bfc2)


# speedup vs baseline: 1.0116x; 1.0116x over previous
import functools

import jax
import jax.numpy as jnp
from jax import lax
from jax.experimental import pallas as pl
from jax.experimental.pallas import tpu as pltpu
from jax.experimental.pallas import tpu_sc as plsc

N = 10000
D = 128
E = 320000
NC = 2
NS = 16
NW = NC * NS
CH = 128
G = 16
K2 = 5
K = K2 * G
EPW = K * CH
E_PAD = NW * EPW
NROW = 10240
ZR = NROW // NS


def _sc_agg_body(x_hbm, src_hbm, dst_hbm, out_hbm, sslab, dslab, bufs,
                 agg_sh, isem, gsems, ssems):
    c = lax.axis_index("c")
    s = lax.axis_index("s")
    wid = s * NC + c

    scope = jax.named_scope
    z16 = jnp.zeros((16,), jnp.float32)

    @pl.loop(0, CH)
    def _zero_row(i):
        for cc in range(D // 16):
            bufs[0][i, pl.ds(cc * 16, 16)] = z16

    zbase = s * ZR
    for t in range(ZR // CH):
        pltpu.async_copy(bufs[0], agg_sh.at[pl.ds(zbase + t * CH, CH)],
                         gsems[1])

    row0 = wid * (K2 * G)
    pltpu.async_copy(src_hbm.at[pl.ds(row0, G)], sslab.at[0], isem)
    pltpu.async_copy(dst_hbm.at[pl.ds(row0, G)], dslab.at[0], isem)
    pltpu.async_copy(src_hbm.at[pl.ds(row0 + G, G)], sslab.at[1], isem)
    pltpu.async_copy(dst_hbm.at[pl.ds(row0 + G, G)], dslab.at[1], isem)
    pltpu.make_async_copy(src_hbm.at[pl.ds(row0, G)], sslab.at[0], isem).wait()
    pltpu.make_async_copy(dst_hbm.at[pl.ds(row0, G)], dslab.at[0], isem).wait()

    with scope("zero_drain"):
        for t in range(ZR // CH):
            pltpu.make_async_copy(bufs[0],
                                  agg_sh.at[pl.ds(zbase + t * CH, CH)],
                                  gsems[1]).wait()
        pltpu.async_copy(x_hbm.at[sslab.at[0, 0]], bufs[0], gsems[0])
        plsc.subcore_barrier()

    def slot(g, p, q, u, first_group, last_group):
        b = u % 2
        pltpu.make_async_copy(x_hbm.at[sslab.at[p, u]], bufs[b],
                              gsems[b]).wait()
        pltpu.async_copy(bufs[b], agg_sh.at[dslab.at[p, u]], ssems[b],
                         add=True)
        if not (first_group and u == 0):
            prev = dslab.at[q, G - 1] if u == 0 else dslab.at[p, u - 1]
            pltpu.make_async_copy(bufs[1 - b], agg_sh.at[prev],
                                  ssems[1 - b]).wait()
        if u == 0 and not first_group and not last_group:
            pltpu.async_copy(src_hbm.at[pl.ds(row0 + (g + 1) * G, G)],
                             sslab.at[q], isem)
            pltpu.async_copy(dst_hbm.at[pl.ds(row0 + (g + 1) * G, G)],
                             dslab.at[q], isem)
        if not (last_group and u == G - 1):
            if u == G - 1:
                pltpu.make_async_copy(src_hbm.at[pl.ds(row0 + (g + 1) * G, G)],
                                      sslab.at[q], isem).wait()
                pltpu.make_async_copy(dst_hbm.at[pl.ds(row0 + (g + 1) * G, G)],
                                      dslab.at[q], isem).wait()
                nxt = sslab.at[q, 0]
            else:
                nxt = sslab.at[p, u + 1]
            pltpu.async_copy(x_hbm.at[nxt], bufs[1 - b], gsems[1 - b])

    def group(g, p, q, first_group=False, last_group=False):
        for u in range(G):
            slot(g, p, q, u, first_group, last_group)

    with scope("edge_pipeline"):
        group(0, 0, 1, first_group=True)

        @pl.loop(1, K2 - 1)
        def _grp(g):
            p = g & 1
            group(g, p, 1 - p)

        group(K2 - 1, (K2 - 1) & 1, 1 - ((K2 - 1) & 1), last_group=True)
        pltpu.make_async_copy(bufs[(G - 1) % 2],
                              agg_sh.at[dslab.at[(K2 - 1) & 1, G - 1]],
                              ssems[(G - 1) % 2]).wait()

    with scope("post_barrier"):
        plsc.subcore_barrier()

    with scope("copyout"):
        nch = ZR // CH
        for t in range(nch):
            r0 = zbase + t * CH
            b = t % 2
            if t >= 2:
                rp = zbase + (t - 2) * CH
                pltpu.make_async_copy(bufs[b], out_hbm.at[c, pl.ds(rp, CH)],
                                      ssems[b]).wait()
            pltpu.sync_copy(agg_sh.at[pl.ds(r0, CH)], bufs[b])
            pltpu.async_copy(bufs[b], out_hbm.at[c, pl.ds(r0, CH)], ssems[b])
        for t in range(nch - 2, nch):
            r0 = zbase + t * CH
            pltpu.make_async_copy(bufs[t % 2], out_hbm.at[c, pl.ds(r0, CH)],
                                  ssems[t % 2]).wait()


@functools.cache
def _make_sc_agg():
    return pl.kernel(
        _sc_agg_body,
        out_type=jax.ShapeDtypeStruct((NC, NROW, D), jnp.float32),
        mesh=plsc.VectorSubcoreMesh(core_axis_name="c", subcore_axis_name="s",
                                    num_cores=NC, num_subcores=NS),
        scratch_types=[
            pltpu.VMEM((2, G, CH), jnp.int32),
            pltpu.VMEM((2, G, CH), jnp.int32),
            [pltpu.VMEM((CH, D), jnp.float32) for _ in range(2)],
            pltpu.VMEM_SHARED((NROW, D), jnp.float32),
            pltpu.SemaphoreType.DMA,
            [pltpu.SemaphoreType.DMA for _ in range(2)],
            [pltpu.SemaphoreType.DMA for _ in range(2)],
        ],
    )


def _sc_agg(x, src_w, dst_w):
    return _make_sc_agg()(x, src_w, dst_w)


_PBR = 320


def _prep_body(ei_ref, src_ref, dst_ref):
    i = pl.program_id(0)
    eloc = (jax.lax.broadcasted_iota(jnp.int32, (_PBR, CH), 0) * CH
            + jax.lax.broadcasted_iota(jnp.int32, (_PBR, CH), 1))
    e = i * (_PBR * CH) + eloc
    real = e < E
    ei = ei_ref[...]
    src_ref[...] = jnp.where(real, ei[0].reshape(_PBR, CH), e % N)
    dst_ref[...] = jnp.where(real, ei[1].reshape(_PBR, CH),
                             N + e % (NROW - N))


def _prep_edges(edge_index):
    nchunk = E_PAD // CH
    return pl.pallas_call(
        _prep_body,
        grid=(nchunk // _PBR,),
        in_specs=[pl.BlockSpec((2, _PBR * CH), lambda i: (0, i))],
        out_specs=[pl.BlockSpec((_PBR, CH), lambda i: (i, 0)),
                   pl.BlockSpec((_PBR, CH), lambda i: (i, 0))],
        out_shape=[jax.ShapeDtypeStruct((nchunk, CH), jnp.int32),
                   jax.ShapeDtypeStruct((nchunk, CH), jnp.int32)],
    )(edge_index)


def _mlp_hidden_body(x_ref, p_ref, wa_ref, ba_ref, wb_ref, bb_ref, o_ref):
    p = p_ref[...]
    h = x_ref[...] + p[0] + p[1]
    t = jnp.maximum(
        jnp.dot(h, wa_ref[...], preferred_element_type=jnp.float32)
        + ba_ref[...], 0.0)
    o_ref[...] = jnp.maximum(
        jnp.dot(t, wb_ref[...], preferred_element_type=jnp.float32)
        + bb_ref[...], 0.0)


def _mlp_final_body(x_ref, p_ref, wa_ref, ba_ref, wb_ref, bb_ref,
                    wfc_ref, bfc_ref, o_ref):
    p = p_ref[...]
    h = x_ref[...] + p[0] + p[1]
    t = jnp.maximum(
        jnp.dot(h, wa_ref[...], preferred_element_type=jnp.float32)
        + ba_ref[...], 0.0)
    t = jnp.maximum(
        jnp.dot(t, wb_ref[...], preferred_element_type=jnp.float32)
        + bb_ref[...], 0.0)
    o_ref[...] = jax.nn.sigmoid(
        jnp.dot(t, wfc_ref[...], preferred_element_type=jnp.float32)
        + bfc_ref[...])


_RB = 2000


def _w_spec(d0, d1):
    return pl.BlockSpec((d0, d1), lambda i: (0, 0))


def _mlp_hidden(x, p, wa, ba, wb, bb):
    return pl.pallas_call(
        _mlp_hidden_body,
        grid=(N // _RB,),
        in_specs=[
            pl.BlockSpec((_RB, D), lambda i: (i, 0)),
            pl.BlockSpec((NC, _RB, D), lambda i: (0, i, 0)),
            _w_spec(D, D), _w_spec(1, D), _w_spec(D, D), _w_spec(1, D),
        ],
        out_specs=pl.BlockSpec((_RB, D), lambda i: (i, 0)),
        out_shape=jax.ShapeDtypeStruct((N, D), jnp.float32),
    )(x, p, wa, ba, wb, bb)


def _mlp_final(x, p, wa, ba, wb, bb, wfc, bfc):
    dout = wfc.shape[1]
    return pl.pallas_call(
        _mlp_final_body,
        grid=(N // _RB,),
        in_specs=[
            pl.BlockSpec((_RB, D), lambda i: (i, 0)),
            pl.BlockSpec((NC, _RB, D), lambda i: (0, i, 0)),
            _w_spec(D, D), _w_spec(1, D), _w_spec(D, D), _w_spec(1, D),
            _w_spec(D, dout), _w_spec(1, dout),
        ],
        out_specs=pl.BlockSpec((_RB, dout), lambda i: (i, 0)),
        out_shape=jax.ShapeDtypeStruct((N, dout), jnp.float32),
    )(x, p, wa, ba, wb, bb, wfc, bfc)


def kernel(x, edge_index, W1a, b1a, W1b, b1b, W2a, b2a, W2b, b2b, Wfc, bfc):
    src_w, dst_w = _prep_edges(edge_index)

    b1a2, b1b2 = b1a.reshape(1, D), b1b.reshape(1, D)
    b2a2, b2b2 = b2a.reshape(1, D), b2b.reshape(1, D)
    bfc2 = bfc.reshape(1, -1)

    p1 = _sc_agg(x, src_w, dst_w)
    h1 = _mlp_hidden(x, p1, W1a, b1a2, W1b, b1b2)
    p2 = _sc_agg(h1, src_w, dst_w)
    return _mlp_final(h1, p2, W2a, b2a2, W2b, b2b2, Wfc, bfc2)
